# Initial kernel scaffold; baseline (speedup 1.0000x reference)
import jax
import jax.numpy as jnp
from jax.experimental import pallas as pl


def kernel(x, edge_index, W_mu, b_mu, W_logstd, b_logstd):
    raise NotImplementedError("write your pallas kernel here")


# speedup vs baseline: 14.4425x; 14.4425x over previous
import functools

import jax
import jax.numpy as jnp
from jax import lax
from jax.experimental import pallas as pl
from jax.experimental.pallas import tpu as pltpu
from jax.experimental.pallas import tpu_sc as plsc

N = 10000
C = 128
E = 320000
NC = 2
NS = 16
NW = NC * NS
CHUNK = 128
CPW = 80
E_PAD = NW * CPW * CHUNK
N_PAD = 10240
ACC_ROWS = N_PAD
RPT = N_PAD // NS
DEG_W = 16

_mesh = plsc.VectorSubcoreMesh(
    core_axis_name="c", subcore_axis_name="s", num_cores=NC, num_subcores=NS
)


def _fill_f32(ref, nrows, ncols, val):
    vec = jnp.full((16,), val, jnp.float32)

    def row(i, carry):
        for k in range(ncols // 16):
            ref[i, pl.ds(k * 16, 16)] = vec
        return carry

    lax.fori_loop(0, nrows, row, 0)


@functools.partial(
    pl.kernel,
    out_type=jax.ShapeDtypeStruct((NC, N_PAD, C), jnp.float32),
    mesh=_mesh,
    scratch_types=[
        pltpu.VMEM((CPW, CHUNK), jnp.int32),
        pltpu.VMEM((CHUNK, DEG_W), jnp.float32),
        pltpu.VMEM((64, DEG_W), jnp.float32),
        pltpu.VMEM((64, C), jnp.float32),
        pltpu.VMEM_SHARED((ACC_ROWS, DEG_W), jnp.float32),
    ],
)
def _deg_kernel(dst_hbm, out_hbm, idx_v, ones_v, buf16, buf128, acc_sh):
    cid = lax.axis_index("c")
    sid = lax.axis_index("s")
    wid = cid * NS + sid
    _fill_f32(ones_v, CHUNK, DEG_W, 1.0)
    _fill_f32(buf16, 64, DEG_W, 0.0)
    for z in range(RPT // 64):
        pltpu.sync_copy(buf16, acc_sh.at[pl.ds(sid * RPT + z * 64, 64)])
    plsc.subcore_barrier()
    pltpu.sync_copy(dst_hbm.at[pl.ds(wid * CPW, CPW)], idx_v)

    def body(j, carry):
        pltpu.sync_copy(ones_v, acc_sh.at[idx_v.at[j]], add=True)
        return carry

    lax.fori_loop(0, CPW, body, 0)
    plsc.subcore_barrier()
    def wb(g, carry):
        pltpu.sync_copy(acc_sh.at[pl.ds(sid * RPT + g * 64, 64)], buf16)

        def rep(i, carry2):
            v = buf16[i]
            for k in range(C // DEG_W):
                buf128[i, pl.ds(k * DEG_W, DEG_W)] = v
            return carry2

        lax.fori_loop(0, 64, rep, 0)
        pltpu.sync_copy(buf128, out_hbm.at[cid, pl.ds(sid * RPT + g * 64, 64)])
        return carry

    lax.fori_loop(0, RPT // 64, wb, 0)


@functools.partial(
    pl.kernel,
    out_type=jax.ShapeDtypeStruct((NC, N_PAD, C), jnp.float32),
    mesh=_mesh,
    scratch_types=[
        pltpu.VMEM((CPW, CHUNK), jnp.int32),
        pltpu.VMEM((CPW, CHUNK), jnp.int32),
        pltpu.VMEM((CHUNK, C), jnp.float32),
        pltpu.VMEM_SHARED((ACC_ROWS, C), jnp.float32),
        pltpu.SemaphoreType.DMA,
    ],
)
def _agg_kernel(y_hbm, src_hbm, dst_hbm, out_hbm, src_v, dst_v, rows_v,
                acc_sh, sem):
    cid = lax.axis_index("c")
    sid = lax.axis_index("s")
    wid = cid * NS + sid
    _fill_f32(rows_v, CHUNK, C, 0.0)
    for k in range(RPT // CHUNK):
        pltpu.sync_copy(rows_v, acc_sh.at[pl.ds(sid * RPT + k * CHUNK, CHUNK)])
    plsc.subcore_barrier()
    pltpu.sync_copy(src_hbm.at[pl.ds(wid * CPW, CPW)], src_v)
    pltpu.sync_copy(dst_hbm.at[pl.ds(wid * CPW, CPW)], dst_v)

    def body(j, carry):
        pltpu.async_copy(y_hbm.at[src_v.at[j]], rows_v, sem).wait()
        pltpu.sync_copy(rows_v, acc_sh.at[dst_v.at[j]], add=True)
        return carry

    lax.fori_loop(0, CPW, body, 0)
    plsc.subcore_barrier()

    def wb(k, carry):
        row0 = sid * RPT + k * CHUNK
        pltpu.sync_copy(acc_sh.at[pl.ds(row0, CHUNK)], rows_v)
        pltpu.sync_copy(rows_v, out_hbm.at[cid, pl.ds(row0, CHUNK)])
        return carry

    lax.fori_loop(0, RPT // CHUNK, wb, 0)


_BR = 1000
_G = N // _BR


def _scale_body(degp_ref, x_ref, y_ref):
    deg = degp_ref[0, :, 0:1] + degp_ref[1, :, 0:1] + 1.0
    y_ref[...] = lax.rsqrt(deg) * x_ref[...]


_scale = pl.pallas_call(
    _scale_body,
    grid=(_G,),
    in_specs=[
        pl.BlockSpec((2, _BR, C), lambda i: (0, i, 0)),
        pl.BlockSpec((_BR, C), lambda i: (i, 0)),
    ],
    out_specs=pl.BlockSpec((_BR, C), lambda i: (i, 0)),
    out_shape=jax.ShapeDtypeStruct((N, C), jnp.float32),
)


def _out_body(degp_ref, accp_ref, y_ref, wmu_ref, bmu_ref, wls_ref, bls_ref,
              mu_ref, ls_ref):
    deg = degp_ref[0, :, 0:1] + degp_ref[1, :, 0:1] + 1.0
    agg = lax.rsqrt(deg) * (accp_ref[0] + accp_ref[1] + y_ref[...])
    mu_ref[...] = jnp.dot(agg, wmu_ref[...],
                          preferred_element_type=jnp.float32) + bmu_ref[...]
    ls_ref[...] = jnp.dot(agg, wls_ref[...],
                          preferred_element_type=jnp.float32) + bls_ref[...]


_outputs = pl.pallas_call(
    _out_body,
    grid=(_G,),
    in_specs=[
        pl.BlockSpec((2, _BR, C), lambda i: (0, i, 0)),
        pl.BlockSpec((2, _BR, C), lambda i: (0, i, 0)),
        pl.BlockSpec((_BR, C), lambda i: (i, 0)),
        pl.BlockSpec((C, C), lambda i: (0, 0)),
        pl.BlockSpec((1, C), lambda i: (0, 0)),
        pl.BlockSpec((C, C), lambda i: (0, 0)),
        pl.BlockSpec((1, C), lambda i: (0, 0)),
    ],
    out_specs=[
        pl.BlockSpec((_BR, C), lambda i: (i, 0)),
        pl.BlockSpec((_BR, C), lambda i: (i, 0)),
    ],
    out_shape=[
        jax.ShapeDtypeStruct((N, C), jnp.float32),
        jax.ShapeDtypeStruct((N, C), jnp.float32),
    ],
)


def kernel(x, edge_index, W_mu, b_mu, W_logstd, b_logstd):
    src = edge_index[0].astype(jnp.int32)
    dst = edge_index[1].astype(jnp.int32)
    pad = E_PAD - E
    src_p = jnp.concatenate([src, jnp.zeros((pad,), jnp.int32)])
    src_p = src_p.reshape(NW * CPW, CHUNK)
    dst_p = jnp.concatenate([dst, jnp.full((pad,), N, jnp.int32)])
    dst_p = dst_p.reshape(NW * CPW, CHUNK)
    degp = _deg_kernel(dst_p)
    y = _scale(degp, x)
    accp = _agg_kernel(y, src_p, dst_p)
    mu, logstd = _outputs(degp, accp, y, W_mu, b_mu.reshape(1, C),
                          W_logstd, b_logstd.reshape(1, C))
    return (mu, logstd)


# speedup vs baseline: 37.8182x; 2.6185x over previous
import functools

import jax
import jax.numpy as jnp
from jax import lax
from jax.experimental import pallas as pl
from jax.experimental.pallas import tpu as pltpu
from jax.experimental.pallas import tpu_sc as plsc

N = 10000
C = 128
E = 320000
NC = 2
NS = 16
NW = NC * NS
CHUNK = 128
CPW = 80
E_PAD = NW * CPW * CHUNK
N_PAD = 10240
ACC_ROWS = N_PAD
RPT = N_PAD // NS
DEG_W = 16

_mesh = plsc.VectorSubcoreMesh(
    core_axis_name="c", subcore_axis_name="s", num_cores=NC, num_subcores=NS
)


def _fill_f32(ref, nrows, ncols, val):
    vec = jnp.full((16,), val, jnp.float32)

    def row(i, carry):
        for k in range(ncols // 16):
            ref[i, pl.ds(k * 16, 16)] = vec
        return carry

    lax.fori_loop(0, nrows, row, 0)


@functools.partial(
    pl.kernel,
    out_type=jax.ShapeDtypeStruct((NC, N_PAD, C), jnp.float32),
    mesh=_mesh,
    scratch_types=[
        pltpu.VMEM((CPW, CHUNK), jnp.int32),
        pltpu.VMEM((CHUNK, DEG_W), jnp.float32),
        pltpu.VMEM((64, DEG_W), jnp.float32),
        pltpu.VMEM((64, C), jnp.float32),
        pltpu.VMEM_SHARED((ACC_ROWS, DEG_W), jnp.float32),
    ],
)
def _deg_kernel(dst_hbm, out_hbm, idx_v, ones_v, buf16, buf128, acc_sh):
    cid = lax.axis_index("c")
    sid = lax.axis_index("s")
    wid = cid * NS + sid
    _fill_f32(ones_v, CHUNK, DEG_W, 1.0)
    _fill_f32(buf16, 64, DEG_W, 0.0)
    for z in range(RPT // 64):
        pltpu.sync_copy(buf16, acc_sh.at[pl.ds(sid * RPT + z * 64, 64)])
    plsc.subcore_barrier()
    pltpu.sync_copy(dst_hbm.at[pl.ds(wid * CPW, CPW)], idx_v)

    def body(j, carry):
        pltpu.sync_copy(ones_v, acc_sh.at[idx_v.at[j]], add=True)
        return carry

    lax.fori_loop(0, CPW, body, 0)
    plsc.subcore_barrier()
    def wb(g, carry):
        pltpu.sync_copy(acc_sh.at[pl.ds(sid * RPT + g * 64, 64)], buf16)

        def rep(i, carry2):
            v = buf16[i]
            for k in range(C // DEG_W):
                buf128[i, pl.ds(k * DEG_W, DEG_W)] = v
            return carry2

        lax.fori_loop(0, 64, rep, 0)
        pltpu.sync_copy(buf128, out_hbm.at[cid, pl.ds(sid * RPT + g * 64, 64)])
        return carry

    lax.fori_loop(0, RPT // 64, wb, 0)


@functools.partial(
    pl.kernel,
    out_type=jax.ShapeDtypeStruct((NC, N_PAD, C), jnp.float32),
    mesh=_mesh,
    scratch_types=[
        pltpu.VMEM((CPW, CHUNK), jnp.int32),
        pltpu.VMEM((CPW, CHUNK), jnp.int32),
        pltpu.VMEM((CHUNK, C), jnp.float32),
        pltpu.VMEM_SHARED((ACC_ROWS, C), jnp.float32),
        pltpu.SemaphoreType.DMA,
    ],
)
def _agg_kernel(y_hbm, src_hbm, dst_hbm, out_hbm, src_v, dst_v, rows_v,
                acc_sh, sem):
    cid = lax.axis_index("c")
    sid = lax.axis_index("s")
    wid = cid * NS + sid
    _fill_f32(rows_v, CHUNK, C, 0.0)
    for k in range(RPT // CHUNK):
        pltpu.sync_copy(rows_v, acc_sh.at[pl.ds(sid * RPT + k * CHUNK, CHUNK)])
    plsc.subcore_barrier()
    pltpu.sync_copy(src_hbm.at[pl.ds(wid * CPW, CPW)], src_v)
    pltpu.sync_copy(dst_hbm.at[pl.ds(wid * CPW, CPW)], dst_v)

    def body(j, carry):
        pltpu.async_copy(y_hbm.at[src_v.at[j]], rows_v, sem).wait()
        pltpu.sync_copy(rows_v, acc_sh.at[dst_v.at[j]], add=True)
        return carry

    lax.fori_loop(0, CPW, body, 0)
    plsc.subcore_barrier()

    def wb(k, carry):
        row0 = sid * RPT + k * CHUNK
        pltpu.sync_copy(acc_sh.at[pl.ds(row0, CHUNK)], rows_v)
        pltpu.sync_copy(rows_v, out_hbm.at[cid, pl.ds(row0, CHUNK)])
        return carry

    lax.fori_loop(0, RPT // CHUNK, wb, 0)


_BR = 1000
_G = N // _BR


def _scale_body(degp_ref, x_ref, y_ref):
    deg = degp_ref[0, :, 0:1] + degp_ref[1, :, 0:1] + 1.0
    y_ref[...] = lax.rsqrt(deg) * x_ref[...]


_scale = pl.pallas_call(
    _scale_body,
    grid=(_G,),
    in_specs=[
        pl.BlockSpec((2, _BR, C), lambda i: (0, i, 0)),
        pl.BlockSpec((_BR, C), lambda i: (i, 0)),
    ],
    out_specs=pl.BlockSpec((_BR, C), lambda i: (i, 0)),
    out_shape=jax.ShapeDtypeStruct((N, C), jnp.float32),
)


def _out_body(degp_ref, accp_ref, y_ref, wmu_ref, bmu_ref, wls_ref, bls_ref,
              mu_ref, ls_ref):
    deg = degp_ref[0, :, 0:1] + degp_ref[1, :, 0:1] + 1.0
    agg = lax.rsqrt(deg) * (accp_ref[0] + accp_ref[1] + y_ref[...])
    mu_ref[...] = jnp.dot(agg, wmu_ref[...],
                          preferred_element_type=jnp.float32) + bmu_ref[...]
    ls_ref[...] = jnp.dot(agg, wls_ref[...],
                          preferred_element_type=jnp.float32) + bls_ref[...]


_outputs = pl.pallas_call(
    _out_body,
    grid=(_G,),
    in_specs=[
        pl.BlockSpec((2, _BR, C), lambda i: (0, i, 0)),
        pl.BlockSpec((2, _BR, C), lambda i: (0, i, 0)),
        pl.BlockSpec((_BR, C), lambda i: (i, 0)),
        pl.BlockSpec((C, C), lambda i: (0, 0)),
        pl.BlockSpec((1, C), lambda i: (0, 0)),
        pl.BlockSpec((C, C), lambda i: (0, 0)),
        pl.BlockSpec((1, C), lambda i: (0, 0)),
    ],
    out_specs=[
        pl.BlockSpec((_BR, C), lambda i: (i, 0)),
        pl.BlockSpec((_BR, C), lambda i: (i, 0)),
    ],
    out_shape=[
        jax.ShapeDtypeStruct((N, C), jnp.float32),
        jax.ShapeDtypeStruct((N, C), jnp.float32),
    ],
)


def kernel(x, edge_index, W_mu, b_mu, W_logstd, b_logstd):
    src = edge_index[0].astype(jnp.int32)
    dst = edge_index[1].astype(jnp.int32)
    pad = E_PAD - E
    pad_iota = jnp.arange(pad, dtype=jnp.int32)
    src_p = jnp.concatenate([src, pad_iota % N])
    src_p = src_p.reshape(NW * CPW, CHUNK)
    dst_p = jnp.concatenate([dst, N + pad_iota % (N_PAD - N)])
    dst_p = dst_p.reshape(NW * CPW, CHUNK)
    degp = _deg_kernel(dst_p)
    y = _scale(degp, x)
    accp = _agg_kernel(y, src_p, dst_p)
    mu, logstd = _outputs(degp, accp, y, W_mu, b_mu.reshape(1, C),
                          W_logstd, b_logstd.reshape(1, C))
    return (mu, logstd)


# speedup vs baseline: 50.7913x; 1.3430x over previous
import functools

import jax
import jax.numpy as jnp
from jax import lax
from jax.experimental import pallas as pl
from jax.experimental.pallas import tpu as pltpu
from jax.experimental.pallas import tpu_sc as plsc

N = 10000
C = 128
E = 320000
NC = 2
NS = 16
NW = NC * NS
CHUNK = 128
CPW = 80
E_PAD = NW * CPW * CHUNK
N_PAD = 10240
ACC_ROWS = N_PAD
RPT = N_PAD // NS
DEG_W = 16

_mesh = plsc.VectorSubcoreMesh(
    core_axis_name="c", subcore_axis_name="s", num_cores=NC, num_subcores=NS
)


def _fill_f32(ref, nrows, ncols, val):
    vec = jnp.full((16,), val, jnp.float32)

    def row(i, carry):
        for k in range(ncols // 16):
            ref[i, pl.ds(k * 16, 16)] = vec
        return carry

    lax.fori_loop(0, nrows, row, 0)


@functools.partial(
    pl.kernel,
    out_type=jax.ShapeDtypeStruct((NC, N_PAD, C), jnp.float32),
    mesh=_mesh,
    scratch_types=[
        pltpu.VMEM((CPW, CHUNK), jnp.int32),
        pltpu.VMEM((CHUNK, DEG_W), jnp.float32),
        pltpu.VMEM((64, DEG_W), jnp.float32),
        pltpu.VMEM((64, C), jnp.float32),
        pltpu.VMEM_SHARED((ACC_ROWS, DEG_W), jnp.float32),
    ],
)
def _deg_kernel(dst_hbm, out_hbm, idx_v, ones_v, buf16, buf128, acc_sh):
    cid = lax.axis_index("c")
    sid = lax.axis_index("s")
    wid = cid * NS + sid
    _fill_f32(ones_v, CHUNK, DEG_W, 1.0)
    _fill_f32(buf16, 64, DEG_W, 0.0)
    for z in range(RPT // 64):
        pltpu.sync_copy(buf16, acc_sh.at[pl.ds(sid * RPT + z * 64, 64)])
    plsc.subcore_barrier()
    pltpu.sync_copy(dst_hbm.at[pl.ds(wid * CPW, CPW)], idx_v)

    def body(j, carry):
        pltpu.sync_copy(ones_v, acc_sh.at[idx_v.at[j]], add=True)
        return carry

    lax.fori_loop(0, CPW, body, 0)
    plsc.subcore_barrier()
    def wb(g, carry):
        pltpu.sync_copy(acc_sh.at[pl.ds(sid * RPT + g * 64, 64)], buf16)

        def rep(i, carry2):
            v = buf16[i]
            for k in range(C // DEG_W):
                buf128[i, pl.ds(k * DEG_W, DEG_W)] = v
            return carry2

        lax.fori_loop(0, 64, rep, 0)
        pltpu.sync_copy(buf128, out_hbm.at[cid, pl.ds(sid * RPT + g * 64, 64)])
        return carry

    lax.fori_loop(0, RPT // 64, wb, 0)


@functools.partial(
    pl.kernel,
    out_type=jax.ShapeDtypeStruct((NC, N_PAD, C), jnp.float32),
    mesh=_mesh,
    scratch_types=[
        pltpu.VMEM((CPW // 2, CHUNK), jnp.int32),
        pltpu.VMEM((CPW // 2, CHUNK), jnp.int32),
        pltpu.VMEM((CHUNK, C), jnp.float32),
        pltpu.VMEM((CHUNK, C), jnp.float32),
        pltpu.VMEM_SHARED((ACC_ROWS, C), jnp.float32),
        pltpu.SemaphoreType.DMA,
        pltpu.SemaphoreType.DMA,
    ],
)
def _agg_kernel(y_hbm, src_hbm, dst_hbm, out_hbm, src_v, dst_v, rows0, rows1,
                acc_sh, sem0, sem1):
    cid = lax.axis_index("c")
    sid = lax.axis_index("s")
    wid = cid * NS + sid
    CPP = CPW // 2
    _fill_f32(rows0, CHUNK, C, 0.0)
    for k in range(RPT // CHUNK):
        pltpu.sync_copy(rows0, acc_sh.at[pl.ds(sid * RPT + k * CHUNK, CHUNK)])
    plsc.subcore_barrier()

    for p in range(2):
        base = wid * CPW + p * CPP
        pltpu.sync_copy(src_hbm.at[pl.ds(base, CPP)], src_v)
        pltpu.sync_copy(dst_hbm.at[pl.ds(base, CPP)], dst_v)
        pltpu.async_copy(y_hbm.at[src_v.at[0]], rows0, sem0)

        def body(t, carry):
            j0 = 2 * t
            j1 = j0 + 1
            pltpu.async_copy(y_hbm.at[src_v.at[j1]], rows1, sem1)
            pltpu.make_async_copy(y_hbm.at[src_v.at[j0]], rows0, sem0).wait()
            pltpu.sync_copy(rows0, acc_sh.at[dst_v.at[j0]], add=True)

            @pl.when(j0 + 2 < CPP)
            def _():
                pltpu.async_copy(y_hbm.at[src_v.at[j0 + 2]], rows0, sem0)

            pltpu.make_async_copy(y_hbm.at[src_v.at[j1]], rows1, sem1).wait()
            pltpu.sync_copy(rows1, acc_sh.at[dst_v.at[j1]], add=True)
            return carry

        lax.fori_loop(0, CPP // 2, body, 0)
    plsc.subcore_barrier()

    def wb(k, carry):
        row0 = sid * RPT + k * CHUNK
        pltpu.sync_copy(acc_sh.at[pl.ds(row0, CHUNK)], rows0)
        pltpu.sync_copy(rows0, out_hbm.at[cid, pl.ds(row0, CHUNK)])
        return carry

    lax.fori_loop(0, RPT // CHUNK, wb, 0)


_BR = 1000
_G = N // _BR


def _scale_body(degp_ref, x_ref, y_ref):
    deg = degp_ref[0, :, 0:1] + degp_ref[1, :, 0:1] + 1.0
    y_ref[...] = lax.rsqrt(deg) * x_ref[...]


_scale = pl.pallas_call(
    _scale_body,
    grid=(_G,),
    in_specs=[
        pl.BlockSpec((2, _BR, C), lambda i: (0, i, 0)),
        pl.BlockSpec((_BR, C), lambda i: (i, 0)),
    ],
    out_specs=pl.BlockSpec((_BR, C), lambda i: (i, 0)),
    out_shape=jax.ShapeDtypeStruct((N, C), jnp.float32),
)


def _out_body(degp_ref, accp_ref, y_ref, wmu_ref, bmu_ref, wls_ref, bls_ref,
              mu_ref, ls_ref):
    deg = degp_ref[0, :, 0:1] + degp_ref[1, :, 0:1] + 1.0
    agg = lax.rsqrt(deg) * (accp_ref[0] + accp_ref[1] + y_ref[...])
    mu_ref[...] = jnp.dot(agg, wmu_ref[...],
                          preferred_element_type=jnp.float32) + bmu_ref[...]
    ls_ref[...] = jnp.dot(agg, wls_ref[...],
                          preferred_element_type=jnp.float32) + bls_ref[...]


_outputs = pl.pallas_call(
    _out_body,
    grid=(_G,),
    in_specs=[
        pl.BlockSpec((2, _BR, C), lambda i: (0, i, 0)),
        pl.BlockSpec((2, _BR, C), lambda i: (0, i, 0)),
        pl.BlockSpec((_BR, C), lambda i: (i, 0)),
        pl.BlockSpec((C, C), lambda i: (0, 0)),
        pl.BlockSpec((1, C), lambda i: (0, 0)),
        pl.BlockSpec((C, C), lambda i: (0, 0)),
        pl.BlockSpec((1, C), lambda i: (0, 0)),
    ],
    out_specs=[
        pl.BlockSpec((_BR, C), lambda i: (i, 0)),
        pl.BlockSpec((_BR, C), lambda i: (i, 0)),
    ],
    out_shape=[
        jax.ShapeDtypeStruct((N, C), jnp.float32),
        jax.ShapeDtypeStruct((N, C), jnp.float32),
    ],
)


def kernel(x, edge_index, W_mu, b_mu, W_logstd, b_logstd):
    src = edge_index[0].astype(jnp.int32)
    dst = edge_index[1].astype(jnp.int32)
    pad = E_PAD - E
    pad_iota = jnp.arange(pad, dtype=jnp.int32)
    src_p = jnp.concatenate([src, pad_iota % N])
    src_p = src_p.reshape(NW * CPW, CHUNK)
    dst_p = jnp.concatenate([dst, N + pad_iota % (N_PAD - N)])
    dst_p = dst_p.reshape(NW * CPW, CHUNK)
    degp = _deg_kernel(dst_p)
    y = _scale(degp, x)
    accp = _agg_kernel(y, src_p, dst_p)
    mu, logstd = _outputs(degp, accp, y, W_mu, b_mu.reshape(1, C),
                          W_logstd, b_logstd.reshape(1, C))
    return (mu, logstd)


# speedup vs baseline: 52.2562x; 1.0288x over previous
import functools

import jax
import jax.numpy as jnp
from jax import lax
from jax.experimental import pallas as pl
from jax.experimental.pallas import tpu as pltpu
from jax.experimental.pallas import tpu_sc as plsc

N = 10000
C = 128
E = 320000
NC = 2
NS = 16
NW = NC * NS
CHUNK = 128
CPW = 80
E_PAD = NW * CPW * CHUNK
N_PAD = 10240
ACC_ROWS = N_PAD
RPT = N_PAD // NS
DEG_W = 16

_mesh = plsc.VectorSubcoreMesh(
    core_axis_name="c", subcore_axis_name="s", num_cores=NC, num_subcores=NS
)


def _fill_f32(ref, nrows, ncols, val):
    vec = jnp.full((16,), val, jnp.float32)

    def row(i, carry):
        for k in range(ncols // 16):
            ref[i, pl.ds(k * 16, 16)] = vec
        return carry

    lax.fori_loop(0, nrows, row, 0)


@functools.partial(
    pl.kernel,
    out_type=jax.ShapeDtypeStruct((NC, N_PAD, C), jnp.float32),
    mesh=_mesh,
    scratch_types=[
        pltpu.VMEM((CPW, CHUNK), jnp.int32),
        pltpu.VMEM((CHUNK, DEG_W), jnp.float32),
        pltpu.VMEM((64, DEG_W), jnp.float32),
        pltpu.VMEM((64, C), jnp.float32),
        pltpu.VMEM_SHARED((ACC_ROWS, DEG_W), jnp.float32),
        pltpu.SemaphoreType.DMA,
    ],
)
def _deg_kernel(dst_hbm, out_hbm, idx_v, ones_v, buf16, buf128, acc_sh, sem):
    cid = lax.axis_index("c")
    sid = lax.axis_index("s")
    wid = cid * NS + sid
    _fill_f32(ones_v, CHUNK, DEG_W, 1.0)
    _fill_f32(buf16, 64, DEG_W, 0.0)
    for z in range(RPT // 64):
        pltpu.sync_copy(buf16, acc_sh.at[pl.ds(sid * RPT + z * 64, 64)])
    plsc.subcore_barrier()
    pltpu.sync_copy(dst_hbm.at[pl.ds(wid * CPW, CPW)], idx_v)

    K = 16

    def group(g, carry):
        def fire(j, carry2):
            pltpu.async_copy(ones_v, acc_sh.at[idx_v.at[g * K + j]], sem,
                             add=True)
            return carry2

        lax.fori_loop(0, K, fire, 0)

        def drain(j, carry2):
            pltpu.make_async_copy(
                ones_v, acc_sh.at[idx_v.at[g * K + j]], sem).wait()
            return carry2

        lax.fori_loop(0, K, drain, 0)
        return carry

    lax.fori_loop(0, CPW // K, group, 0)
    plsc.subcore_barrier()
    def wb(g, carry):
        pltpu.sync_copy(acc_sh.at[pl.ds(sid * RPT + g * 64, 64)], buf16)

        def rep(i, carry2):
            v = buf16[i]
            for k in range(C // DEG_W):
                buf128[i, pl.ds(k * DEG_W, DEG_W)] = v
            return carry2

        lax.fori_loop(0, 64, rep, 0)
        pltpu.sync_copy(buf128, out_hbm.at[cid, pl.ds(sid * RPT + g * 64, 64)])
        return carry

    lax.fori_loop(0, RPT // 64, wb, 0)


@functools.partial(
    pl.kernel,
    out_type=jax.ShapeDtypeStruct((NC, N_PAD, C), jnp.float32),
    mesh=_mesh,
    scratch_types=[
        pltpu.VMEM((CPW // 2, CHUNK), jnp.int32),
        pltpu.VMEM((CPW // 2, CHUNK), jnp.int32),
        pltpu.VMEM((CHUNK, C), jnp.float32),
        pltpu.VMEM((CHUNK, C), jnp.float32),
        pltpu.VMEM_SHARED((ACC_ROWS, C), jnp.float32),
        pltpu.SemaphoreType.DMA,
        pltpu.SemaphoreType.DMA,
    ],
)
def _agg_kernel(y_hbm, src_hbm, dst_hbm, out_hbm, src_v, dst_v, rows0, rows1,
                acc_sh, sem0, sem1):
    cid = lax.axis_index("c")
    sid = lax.axis_index("s")
    wid = cid * NS + sid
    CPP = CPW // 2
    _fill_f32(rows0, CHUNK, C, 0.0)
    for k in range(RPT // CHUNK):
        pltpu.sync_copy(rows0, acc_sh.at[pl.ds(sid * RPT + k * CHUNK, CHUNK)])
    plsc.subcore_barrier()

    for p in range(2):
        base = wid * CPW + p * CPP
        pltpu.sync_copy(src_hbm.at[pl.ds(base, CPP)], src_v)
        pltpu.sync_copy(dst_hbm.at[pl.ds(base, CPP)], dst_v)
        pltpu.async_copy(y_hbm.at[src_v.at[0]], rows0, sem0)

        def body(t, carry):
            j0 = 2 * t
            j1 = j0 + 1
            pltpu.async_copy(y_hbm.at[src_v.at[j1]], rows1, sem1)
            pltpu.make_async_copy(y_hbm.at[src_v.at[j0]], rows0, sem0).wait()
            pltpu.sync_copy(rows0, acc_sh.at[dst_v.at[j0]], add=True)

            @pl.when(j0 + 2 < CPP)
            def _():
                pltpu.async_copy(y_hbm.at[src_v.at[j0 + 2]], rows0, sem0)

            pltpu.make_async_copy(y_hbm.at[src_v.at[j1]], rows1, sem1).wait()
            pltpu.sync_copy(rows1, acc_sh.at[dst_v.at[j1]], add=True)
            return carry

        lax.fori_loop(0, CPP // 2, body, 0)
    plsc.subcore_barrier()

    NG = RPT // CHUNK
    bufs = (rows0, rows1)
    pltpu.async_copy(acc_sh.at[pl.ds(sid * RPT, CHUNK)], rows0, sem0)
    for k in range(NG):
        b = bufs[k % 2]
        row0 = sid * RPT + k * CHUNK
        pltpu.make_async_copy(acc_sh.at[pl.ds(row0, CHUNK)], b, sem0).wait()
        if k + 1 < NG:
            pltpu.async_copy(
                acc_sh.at[pl.ds(row0 + CHUNK, CHUNK)], bufs[(k + 1) % 2], sem0)
        pltpu.sync_copy(b, out_hbm.at[cid, pl.ds(row0, CHUNK)])


_BR = 1000
_G = N // _BR


def _scale_body(degp_ref, x_ref, y_ref):
    deg = degp_ref[0, :, 0:1] + degp_ref[1, :, 0:1] + 1.0
    y_ref[...] = lax.rsqrt(deg) * x_ref[...]


_scale = pl.pallas_call(
    _scale_body,
    grid=(_G,),
    in_specs=[
        pl.BlockSpec((2, _BR, C), lambda i: (0, i, 0)),
        pl.BlockSpec((_BR, C), lambda i: (i, 0)),
    ],
    out_specs=pl.BlockSpec((_BR, C), lambda i: (i, 0)),
    out_shape=jax.ShapeDtypeStruct((N, C), jnp.float32),
)


def _out_body(degp_ref, accp_ref, y_ref, wmu_ref, bmu_ref, wls_ref, bls_ref,
              mu_ref, ls_ref):
    deg = degp_ref[0, :, 0:1] + degp_ref[1, :, 0:1] + 1.0
    agg = lax.rsqrt(deg) * (accp_ref[0] + accp_ref[1] + y_ref[...])
    mu_ref[...] = jnp.dot(agg, wmu_ref[...],
                          preferred_element_type=jnp.float32) + bmu_ref[...]
    ls_ref[...] = jnp.dot(agg, wls_ref[...],
                          preferred_element_type=jnp.float32) + bls_ref[...]


_outputs = pl.pallas_call(
    _out_body,
    grid=(_G,),
    in_specs=[
        pl.BlockSpec((2, _BR, C), lambda i: (0, i, 0)),
        pl.BlockSpec((2, _BR, C), lambda i: (0, i, 0)),
        pl.BlockSpec((_BR, C), lambda i: (i, 0)),
        pl.BlockSpec((C, C), lambda i: (0, 0)),
        pl.BlockSpec((1, C), lambda i: (0, 0)),
        pl.BlockSpec((C, C), lambda i: (0, 0)),
        pl.BlockSpec((1, C), lambda i: (0, 0)),
    ],
    out_specs=[
        pl.BlockSpec((_BR, C), lambda i: (i, 0)),
        pl.BlockSpec((_BR, C), lambda i: (i, 0)),
    ],
    out_shape=[
        jax.ShapeDtypeStruct((N, C), jnp.float32),
        jax.ShapeDtypeStruct((N, C), jnp.float32),
    ],
)


def kernel(x, edge_index, W_mu, b_mu, W_logstd, b_logstd):
    src = edge_index[0].astype(jnp.int32)
    dst = edge_index[1].astype(jnp.int32)
    pad = E_PAD - E
    pad_iota = jnp.arange(pad, dtype=jnp.int32)
    src_p = jnp.concatenate([src, pad_iota % N])
    src_p = src_p.reshape(NW * CPW, CHUNK)
    dst_p = jnp.concatenate([dst, N + pad_iota % (N_PAD - N)])
    dst_p = dst_p.reshape(NW * CPW, CHUNK)
    degp = _deg_kernel(dst_p)
    y = _scale(degp, x)
    accp = _agg_kernel(y, src_p, dst_p)
    mu, logstd = _outputs(degp, accp, y, W_mu, b_mu.reshape(1, C),
                          W_logstd, b_logstd.reshape(1, C))
    return (mu, logstd)
